# baseline (speedup 1.0000x reference)
import functools

import jax
import jax.numpy as jnp
from jax import lax
from jax.experimental import pallas as pl
from jax.experimental.pallas import tpu as pltpu
from jax.experimental.pallas import tpu_sc as plsc

B = 16384
EMB = 64
VOCAB = 1000

_NC = 2
_NS = 16
_NW = _NC * _NS
_BPW = B // _NW
_CHUNK = 128
_NCHUNK = _BPW // _CHUNK


def _gather_body(user_table, user_ids, item_table, item_ids,
                 u_out, i_out,
                 uidx_v, iidx_v, urows_v, irows_v, sem):
    wid = lax.axis_index("s") * _NC + lax.axis_index("c")
    base = wid * _BPW
    pltpu.sync_copy(user_ids.at[wid], uidx_v)
    pltpu.sync_copy(item_ids.at[wid], iidx_v)
    copies = []
    for c in range(_NCHUNK):
        copies.append(pltpu.async_copy(
            user_table.at[uidx_v.at[c]],
            urows_v.at[pl.ds(c * _CHUNK, _CHUNK)], sem))
        copies.append(pltpu.async_copy(
            item_table.at[iidx_v.at[c]],
            irows_v.at[pl.ds(c * _CHUNK, _CHUNK)], sem))
    for cp in copies:
        cp.wait()
    pltpu.sync_copy(urows_v, u_out.at[pl.ds(base, _BPW)])
    pltpu.sync_copy(irows_v, i_out.at[pl.ds(base, _BPW)])


@functools.lru_cache(maxsize=1)
def _make_gather():
    return pl.kernel(
        _gather_body,
        mesh=plsc.VectorSubcoreMesh(core_axis_name="c", subcore_axis_name="s"),
        out_type=[
            jax.ShapeDtypeStruct((B, EMB), jnp.float32),
            jax.ShapeDtypeStruct((B, EMB), jnp.float32),
        ],
        scratch_types=[
            pltpu.VMEM((_NCHUNK, _CHUNK), jnp.int32),
            pltpu.VMEM((_NCHUNK, _CHUNK), jnp.int32),
            pltpu.VMEM((_BPW, EMB), jnp.float32),
            pltpu.VMEM((_BPW, EMB), jnp.float32),
            pltpu.SemaphoreType.DMA,
        ],
        compiler_params=pltpu.CompilerParams(use_tc_tiling_on_sc=False),
    )


_BM = 512


def _mm_body(u_ref, i_ref, wu_ref, wi_ref, o_ref):
    o_ref[...] = (
        jnp.dot(u_ref[...], wu_ref[...], preferred_element_type=jnp.float32)
        + jnp.dot(i_ref[...], wi_ref[...], preferred_element_type=jnp.float32)
    )


@functools.partial(jax.jit)
def _matmul(u_emb, i_emb, wu_t, wi_t):
    return pl.pallas_call(
        _mm_body,
        grid=(B // _BM,),
        in_specs=[
            pl.BlockSpec((_BM, EMB), lambda m: (m, 0)),
            pl.BlockSpec((_BM, EMB), lambda m: (m, 0)),
            pl.BlockSpec((EMB, VOCAB), lambda m: (0, 0)),
            pl.BlockSpec((EMB, VOCAB), lambda m: (0, 0)),
        ],
        out_specs=pl.BlockSpec((_BM, VOCAB), lambda m: (m, 0)),
        out_shape=jax.ShapeDtypeStruct((B, VOCAB), jnp.float32),
    )(u_emb, i_emb, wu_t, wi_t)


def kernel(attr_item, attr_tf_item, attr_lens_item, item_ids, attr_user,
           attr_tf_user, attr_lens_user, user_ids, pos_targets, pos_lens,
           neg_targets, neg_lens, user_table, item_table, W_user, W_item):
    uids = user_ids.astype(jnp.int32).reshape(_NW, _NCHUNK, _CHUNK)
    iids = item_ids.astype(jnp.int32).reshape(_NW, _NCHUNK, _CHUNK)
    u_emb, i_emb = _make_gather()(user_table, uids, item_table, iids)
    logits = _matmul(u_emb, i_emb, W_user.T, W_item.T)
    return (logits, None, None)


# speedup vs baseline: 1.0040x; 1.0040x over previous
import functools

import jax
import jax.numpy as jnp
from jax import lax
from jax.experimental import pallas as pl
from jax.experimental.pallas import tpu as pltpu
from jax.experimental.pallas import tpu_sc as plsc

B = 16384
EMB = 64
VOCAB = 1000
VIEW_W = 2 * EMB

_NC = 2
_NS = 16
_NW = _NC * _NS
_BPW = B // _NW
_CHUNK = 128
_NCHUNK = _BPW // _CHUNK


def _gather_body(user_view, uidx_hbm, item_view, iidx_hbm,
                 u_out, i_out,
                 idx_v, rows_v, sem):
    wid = lax.axis_index("s") * _NC + lax.axis_index("c")
    base = wid * _BPW
    for tbl, idx_hbm, out in ((user_view, uidx_hbm, u_out),
                              (item_view, iidx_hbm, i_out)):
        pltpu.sync_copy(idx_hbm.at[wid], idx_v)
        copies = []
        for c in range(_NCHUNK):
            copies.append(pltpu.async_copy(
                tbl.at[idx_v.at[c]],
                rows_v.at[pl.ds(c * _CHUNK, _CHUNK)], sem))
        for cp in copies:
            cp.wait()
        pltpu.sync_copy(rows_v, out.at[pl.ds(base, _BPW)])


@functools.lru_cache(maxsize=1)
def _make_gather():
    return pl.kernel(
        _gather_body,
        mesh=plsc.VectorSubcoreMesh(core_axis_name="c", subcore_axis_name="s"),
        out_type=[
            jax.ShapeDtypeStruct((B, VIEW_W), jnp.float32),
            jax.ShapeDtypeStruct((B, VIEW_W), jnp.float32),
        ],
        scratch_types=[
            pltpu.VMEM((_NCHUNK, _CHUNK), jnp.int32),
            pltpu.VMEM((_BPW, VIEW_W), jnp.float32),
            pltpu.SemaphoreType.DMA,
        ],
    )


_BM = 512


def _mm_body(pp_ref, u_ref, i_ref, wu_ref, wi_ref, o_ref):
    lane = lax.broadcasted_iota(jnp.int32, (_BM, VIEW_W), 1)
    low = lane < EMB
    pp = pp_ref[...]
    u_odd = (pp & 1) == 1
    i_odd = ((pp >> 1) & 1) == 1
    u_mask = jnp.where(low != u_odd, 1.0, 0.0)
    i_mask = jnp.where(low != i_odd, 1.0, 0.0)
    o_ref[...] = (
        jnp.dot(u_ref[...] * u_mask, wu_ref[...],
                preferred_element_type=jnp.float32)
        + jnp.dot(i_ref[...] * i_mask, wi_ref[...],
                  preferred_element_type=jnp.float32)
    )


def _matmul(pp, u2, i2, w2u, w2i):
    return pl.pallas_call(
        _mm_body,
        grid=(B // _BM,),
        in_specs=[
            pl.BlockSpec((_BM, 1), lambda m: (m, 0)),
            pl.BlockSpec((_BM, VIEW_W), lambda m: (m, 0)),
            pl.BlockSpec((_BM, VIEW_W), lambda m: (m, 0)),
            pl.BlockSpec((VIEW_W, VOCAB), lambda m: (0, 0)),
            pl.BlockSpec((VIEW_W, VOCAB), lambda m: (0, 0)),
        ],
        out_specs=pl.BlockSpec((_BM, VOCAB), lambda m: (m, 0)),
        out_shape=jax.ShapeDtypeStruct((B, VOCAB), jnp.float32),
    )(pp, u2, i2, w2u, w2i)


def kernel(attr_item, attr_tf_item, attr_lens_item, item_ids, attr_user,
           attr_tf_user, attr_lens_user, user_ids, pos_targets, pos_lens,
           neg_targets, neg_lens, user_table, item_table, W_user, W_item):
    uids = user_ids.astype(jnp.int32)
    iids = item_ids.astype(jnp.int32)
    user_view = user_table.reshape(user_table.shape[0] // 2, VIEW_W)
    item_view = item_table.reshape(item_table.shape[0] // 2, VIEW_W)
    uidx_h = (uids >> 1).reshape(_NW, _NCHUNK, _CHUNK)
    iidx_h = (iids >> 1).reshape(_NW, _NCHUNK, _CHUNK)
    u2, i2 = _make_gather()(user_view, uidx_h, item_view, iidx_h)
    w2u = jnp.concatenate([W_user.T, W_user.T], axis=0)
    w2i = jnp.concatenate([W_item.T, W_item.T], axis=0)
    pp = ((uids & 1) + 2 * (iids & 1)).reshape(B, 1)
    logits = _matmul(pp, u2, i2, w2u, w2i)
    return (logits, None, None)


# speedup vs baseline: 1.5094x; 1.5033x over previous
import functools

import jax
import jax.numpy as jnp
from jax import lax
from jax.experimental import pallas as pl
from jax.experimental.pallas import tpu as pltpu
from jax.experimental.pallas import tpu_sc as plsc

B = 16384
EMB = 64
VOCAB = 1000

_NC = 2
_NS = 16
_NW = _NC * _NS
_BPW = B // _NW


def _gather_body(user_table, uidx_hbm, item_table, iidx_hbm,
                 u_out, i_out,
                 idx_s, idx_v, rows_v, sem):
    wid = lax.axis_index("s") * _NC + lax.axis_index("c")
    base = wid * _BPW
    for tbl, idx_hbm, out in ((user_table, uidx_hbm, u_out),
                              (item_table, iidx_hbm, i_out)):
        pltpu.sync_copy(idx_hbm.at[wid], idx_v)

        def step(g, _):
            v = idx_v[pl.ds(g * 16, 16)]
            for l in range(16):
                pltpu.async_copy(tbl.at[v[l]], rows_v.at[g * 16 + l], sem)
            return 0

        lax.fori_loop(0, _BPW // 16, step, 0)
        pltpu.make_async_copy(tbl.at[pl.ds(0, _BPW)], rows_v, sem).wait()
        pltpu.sync_copy(rows_v, out.at[pl.ds(base, _BPW)])


@functools.lru_cache(maxsize=1)
def _make_gather():
    return pl.kernel(
        _gather_body,
        mesh=plsc.VectorSubcoreMesh(core_axis_name="c", subcore_axis_name="s"),
        out_type=[
            jax.ShapeDtypeStruct((B, EMB), jnp.float32),
            jax.ShapeDtypeStruct((B, EMB), jnp.float32),
        ],
        scratch_types=[
            pltpu.SMEM((_BPW,), jnp.int32),
            pltpu.VMEM((_BPW,), jnp.int32),
            pltpu.VMEM((_BPW, EMB), jnp.float32),
            pltpu.SemaphoreType.DMA,
        ],
    )


_BM = 512


def _mm_body(u_ref, i_ref, wu_ref, wi_ref, o_ref):
    o_ref[...] = (
        jnp.dot(u_ref[...], wu_ref[...], preferred_element_type=jnp.float32)
        + jnp.dot(i_ref[...], wi_ref[...], preferred_element_type=jnp.float32)
    )


def _matmul(u_emb, i_emb, wu_t, wi_t):
    return pl.pallas_call(
        _mm_body,
        grid=(B // _BM,),
        in_specs=[
            pl.BlockSpec((_BM, EMB), lambda m: (m, 0)),
            pl.BlockSpec((_BM, EMB), lambda m: (m, 0)),
            pl.BlockSpec((EMB, VOCAB), lambda m: (0, 0)),
            pl.BlockSpec((EMB, VOCAB), lambda m: (0, 0)),
        ],
        out_specs=pl.BlockSpec((_BM, VOCAB), lambda m: (m, 0)),
        out_shape=jax.ShapeDtypeStruct((B, VOCAB), jnp.float32),
    )(u_emb, i_emb, wu_t, wi_t)


def kernel(attr_item, attr_tf_item, attr_lens_item, item_ids, attr_user,
           attr_tf_user, attr_lens_user, user_ids, pos_targets, pos_lens,
           neg_targets, neg_lens, user_table, item_table, W_user, W_item):
    uids = user_ids.astype(jnp.int32).reshape(_NW, _BPW)
    iids = item_ids.astype(jnp.int32).reshape(_NW, _BPW)
    u_emb, i_emb = _make_gather()(user_table, uids, item_table, iids)
    logits = _matmul(u_emb, i_emb, W_user.T, W_item.T)
    return (logits, None, None)


# speedup vs baseline: 2.0732x; 1.3735x over previous
import functools

import jax
import jax.numpy as jnp
from jax import lax
from jax.experimental import pallas as pl
from jax.experimental.pallas import tpu as pltpu
from jax.experimental.pallas import tpu_sc as plsc

B = 16384
EMB = 64
VOCAB = 1000
NROWS = 1000000
VIEW_W = 2 * EMB

_BN = 8192
_NBLK = 62
H = _BN * _NBLK
_IN_BLOCKS = (NROWS + _BN - 1) // _BN - 1

_NC = 2
_NS = 16
_NW = _NC * _NS
_BPW = B // _NW


def _rp_body(a_ref, b_ref, o_ref):
    o_ref[:, 0:EMB] = jnp.transpose(a_ref[...])
    o_ref[:, EMB:VIEW_W] = jnp.transpose(b_ref[...])


def _repack(tbl_t):
    return pl.pallas_call(
        _rp_body,
        grid=(_NBLK,),
        in_specs=[
            pl.BlockSpec((EMB, _BN), lambda n: (0, n)),
            pl.BlockSpec((EMB, _BN),
                         lambda n: (0, jnp.minimum(n + _NBLK, _IN_BLOCKS))),
        ],
        out_specs=pl.BlockSpec((_BN, VIEW_W), lambda n: (n, 0)),
        out_shape=jax.ShapeDtypeStruct((H, VIEW_W), jnp.float32),
    )(tbl_t, tbl_t)


def _gather_body(user_tbl, uidx_hbm, item_tbl, iidx_hbm,
                 u_out, i_out,
                 idx_v, rows_v, sem):
    wid = lax.axis_index("s") * _NC + lax.axis_index("c")
    base = wid * _BPW
    for tbl, idx_hbm, out in ((user_tbl, uidx_hbm, u_out),
                              (item_tbl, iidx_hbm, i_out)):
        pltpu.sync_copy(idx_hbm.at[wid], idx_v)

        def step(g, _):
            v = idx_v[pl.ds(g * 16, 16)]
            for l in range(16):
                pltpu.async_copy(tbl.at[v[l]], rows_v.at[g * 16 + l], sem)
            return 0

        lax.fori_loop(0, _BPW // 16, step, 0)
        pltpu.make_async_copy(out.at[pl.ds(0, _BPW)], rows_v, sem).wait()
        pltpu.sync_copy(rows_v, out.at[pl.ds(base, _BPW)])


@functools.lru_cache(maxsize=1)
def _make_gather():
    return pl.kernel(
        _gather_body,
        mesh=plsc.VectorSubcoreMesh(core_axis_name="c", subcore_axis_name="s"),
        out_type=[
            jax.ShapeDtypeStruct((B, VIEW_W), jnp.float32),
            jax.ShapeDtypeStruct((B, VIEW_W), jnp.float32),
        ],
        scratch_types=[
            pltpu.VMEM((_BPW,), jnp.int32),
            pltpu.VMEM((_BPW, VIEW_W), jnp.float32),
            pltpu.SemaphoreType.DMA,
        ],
    )


_BM = 512


def _mm_body(pp_ref, u_ref, i_ref, wu_ref, wi_ref, o_ref):
    lane = lax.broadcasted_iota(jnp.int32, (_BM, VIEW_W), 1)
    low = lane < EMB
    pp = pp_ref[...]
    u_hi = (pp & 1) == 1
    i_hi = ((pp >> 1) & 1) == 1
    u_mask = jnp.where(low != u_hi, 1.0, 0.0)
    i_mask = jnp.where(low != i_hi, 1.0, 0.0)
    o_ref[...] = (
        jnp.dot(u_ref[...] * u_mask, wu_ref[...],
                preferred_element_type=jnp.float32)
        + jnp.dot(i_ref[...] * i_mask, wi_ref[...],
                  preferred_element_type=jnp.float32)
    )


def _matmul(pp, u2, i2, w2u, w2i):
    return pl.pallas_call(
        _mm_body,
        grid=(B // _BM,),
        in_specs=[
            pl.BlockSpec((_BM, 1), lambda m: (m, 0)),
            pl.BlockSpec((_BM, VIEW_W), lambda m: (m, 0)),
            pl.BlockSpec((_BM, VIEW_W), lambda m: (m, 0)),
            pl.BlockSpec((VIEW_W, VOCAB), lambda m: (0, 0)),
            pl.BlockSpec((VIEW_W, VOCAB), lambda m: (0, 0)),
        ],
        out_specs=pl.BlockSpec((_BM, VOCAB), lambda m: (m, 0)),
        out_shape=jax.ShapeDtypeStruct((B, VOCAB), jnp.float32),
    )(pp, u2, i2, w2u, w2i)


def kernel(attr_item, attr_tf_item, attr_lens_item, item_ids, attr_user,
           attr_tf_user, attr_lens_user, user_ids, pos_targets, pos_lens,
           neg_targets, neg_lens, user_table, item_table, W_user, W_item):
    uids = user_ids.astype(jnp.int32)
    iids = item_ids.astype(jnp.int32)
    u_hi = (uids >= H).astype(jnp.int32)
    i_hi = (iids >= H).astype(jnp.int32)
    uv = (uids - u_hi * H).reshape(_NW, _BPW)
    iv = (iids - i_hi * H).reshape(_NW, _BPW)
    u_tbl = _repack(user_table.T)
    i_tbl = _repack(item_table.T)
    u2, i2 = _make_gather()(u_tbl, uv, i_tbl, iv)
    pp = (u_hi + 2 * i_hi).reshape(B, 1)
    w2u = jnp.concatenate([W_user.T, W_user.T], axis=0)
    w2i = jnp.concatenate([W_item.T, W_item.T], axis=0)
    logits = _matmul(pp, u2, i2, w2u, w2i)
    return (logits, None, None)


# speedup vs baseline: 2.5457x; 1.2280x over previous
import functools

import jax
import jax.numpy as jnp
from jax import lax
from jax.experimental import pallas as pl
from jax.experimental.pallas import tpu as pltpu
from jax.experimental.pallas import tpu_sc as plsc

B = 16384
EMB = 64
VOCAB = 1000
NROWS = 1000000
VIEW_W = 2 * EMB

_BN = 8192
_NBLK = 62
H = _BN * _NBLK
_IN_BLOCKS = (NROWS + _BN - 1) // _BN - 1

_NC = 2
_NS = 16
_NW = _NC * _NS
_BPW = B // _NW


def _rp_body(a_ref, b_ref, o_ref):
    a16 = jax.lax.bitcast_convert_type(
        jnp.transpose(a_ref[...]).astype(jnp.bfloat16), jnp.uint16)
    b16 = jax.lax.bitcast_convert_type(
        jnp.transpose(b_ref[...]).astype(jnp.bfloat16), jnp.uint16)
    word = a16.astype(jnp.uint32) | (b16.astype(jnp.uint32) << 16)
    o_ref[...] = jax.lax.bitcast_convert_type(word, jnp.float32)


def _repack(tbl_t):
    return pl.pallas_call(
        _rp_body,
        grid=(_NBLK,),
        in_specs=[
            pl.BlockSpec((EMB, _BN), lambda n: (0, n)),
            pl.BlockSpec((EMB, _BN),
                         lambda n: (0, jnp.minimum(n + _NBLK, _IN_BLOCKS))),
        ],
        out_specs=pl.BlockSpec((_BN, EMB), lambda n: (n, 0)),
        out_shape=jax.ShapeDtypeStruct((H, EMB), jnp.float32),
    )(tbl_t, tbl_t)


def _gather_body(user_tbl, uidx_hbm, item_tbl, iidx_hbm,
                 u_out, i_out,
                 idx_v, rows_v, sem):
    wid = lax.axis_index("s") * _NC + lax.axis_index("c")
    base = wid * _BPW
    for tbl, idx_hbm, out in ((user_tbl, uidx_hbm, u_out),
                              (item_tbl, iidx_hbm, i_out)):
        pltpu.sync_copy(idx_hbm.at[wid], idx_v)

        def step(g, _):
            v = idx_v[pl.ds(g * 16, 16)]
            for l in range(16):
                pltpu.async_copy(tbl.at[v[l]], rows_v.at[g * 16 + l], sem)
            return 0

        lax.fori_loop(0, _BPW // 16, step, 0)
        pltpu.make_async_copy(out.at[pl.ds(0, _BPW)], rows_v, sem).wait()
        pltpu.sync_copy(rows_v, out.at[pl.ds(base, _BPW)])


@functools.lru_cache(maxsize=1)
def _make_gather():
    return pl.kernel(
        _gather_body,
        mesh=plsc.VectorSubcoreMesh(core_axis_name="c", subcore_axis_name="s"),
        out_type=[
            jax.ShapeDtypeStruct((B, EMB), jnp.float32),
            jax.ShapeDtypeStruct((B, EMB), jnp.float32),
        ],
        scratch_types=[
            pltpu.VMEM((_BPW,), jnp.int32),
            pltpu.VMEM((_BPW, EMB), jnp.float32),
            pltpu.SemaphoreType.DMA,
        ],
    )


_BM = 512


def _unpack_half(words_f32, take_hi):
    w = jax.lax.bitcast_convert_type(words_f32, jnp.uint32)
    lo = w << 16
    hi = w & jnp.uint32(0xFFFF0000)
    return jax.lax.bitcast_convert_type(
        jnp.where(take_hi, hi, lo), jnp.float32)


def _mm_body(pp_ref, u_ref, i_ref, wu_ref, wi_ref, o_ref):
    pp = pp_ref[...]
    u_hi = (pp & 1) == 1
    i_hi = ((pp >> 1) & 1) == 1
    u = _unpack_half(u_ref[...], u_hi)
    i = _unpack_half(i_ref[...], i_hi)
    o_ref[...] = (
        jnp.dot(u, wu_ref[...], preferred_element_type=jnp.float32)
        + jnp.dot(i, wi_ref[...], preferred_element_type=jnp.float32)
    )


def _matmul(pp, u2, i2, w2u, w2i):
    return pl.pallas_call(
        _mm_body,
        grid=(B // _BM,),
        in_specs=[
            pl.BlockSpec((_BM, 1), lambda m: (m, 0)),
            pl.BlockSpec((_BM, EMB), lambda m: (m, 0)),
            pl.BlockSpec((_BM, EMB), lambda m: (m, 0)),
            pl.BlockSpec((EMB, VOCAB), lambda m: (0, 0)),
            pl.BlockSpec((EMB, VOCAB), lambda m: (0, 0)),
        ],
        out_specs=pl.BlockSpec((_BM, VOCAB), lambda m: (m, 0)),
        out_shape=jax.ShapeDtypeStruct((B, VOCAB), jnp.float32),
    )(pp, u2, i2, w2u, w2i)


def kernel(attr_item, attr_tf_item, attr_lens_item, item_ids, attr_user,
           attr_tf_user, attr_lens_user, user_ids, pos_targets, pos_lens,
           neg_targets, neg_lens, user_table, item_table, W_user, W_item):
    uids = user_ids.astype(jnp.int32)
    iids = item_ids.astype(jnp.int32)
    u_hi = (uids >= H).astype(jnp.int32)
    i_hi = (iids >= H).astype(jnp.int32)
    uv = (uids - u_hi * H).reshape(_NW, _BPW)
    iv = (iids - i_hi * H).reshape(_NW, _BPW)
    u_tbl = _repack(user_table.T)
    i_tbl = _repack(item_table.T)
    u2, i2 = _make_gather()(u_tbl, uv, i_tbl, iv)
    pp = (u_hi + 2 * i_hi).reshape(B, 1)
    logits = _matmul(pp, u2, i2, W_user.T, W_item.T)
    return (logits, None, None)


# speedup vs baseline: 2.8906x; 1.1355x over previous
import functools

import jax
import jax.numpy as jnp
from jax import lax
from jax.experimental import pallas as pl
from jax.experimental.pallas import tpu as pltpu
from jax.experimental.pallas import tpu_sc as plsc

B = 16384
EMB = 64
VOCAB = 1000
NROWS = 1000000
VIEW_W = 2 * EMB

_BN = 8192
_NBLK = 62
H = _BN * _NBLK
_IN_BLOCKS = (NROWS + _BN - 1) // _BN - 1

_NC = 2
_NS = 16
_NW = _NC * _NS
_BPW = B // _NW


def _rp_body(a_ref, b_ref, o_ref):
    a16 = jax.lax.bitcast_convert_type(
        jnp.transpose(a_ref[...]).astype(jnp.bfloat16), jnp.uint16)
    b16 = jax.lax.bitcast_convert_type(
        jnp.transpose(b_ref[...]).astype(jnp.bfloat16), jnp.uint16)
    word = a16.astype(jnp.uint32) | (b16.astype(jnp.uint32) << 16)
    o_ref[...] = jax.lax.bitcast_convert_type(word, jnp.float32)


def _repack(tbl_t):
    return pl.pallas_call(
        _rp_body,
        grid=(_NBLK,),
        in_specs=[
            pl.BlockSpec((EMB, _BN), lambda n: (0, n)),
            pl.BlockSpec((EMB, _BN),
                         lambda n: (0, jnp.minimum(n + _NBLK, _IN_BLOCKS))),
        ],
        out_specs=pl.BlockSpec((_BN, EMB), lambda n: (n, 0)),
        out_shape=jax.ShapeDtypeStruct((H, EMB), jnp.float32),
    )(tbl_t, tbl_t)


def _gather_body(user_tbl, uidx_hbm, item_tbl, iidx_hbm,
                 u_out, i_out,
                 idx_v, rows_v, sem):
    wid = lax.axis_index("s") * _NC + lax.axis_index("c")
    base = wid * _BPW
    for tbl, idx_hbm, out in ((user_tbl, uidx_hbm, u_out),
                              (item_tbl, iidx_hbm, i_out)):
        pltpu.sync_copy(idx_hbm.at[wid], idx_v)

        def step(g, _):
            v = idx_v[pl.ds(g * 16, 16)]
            for l in range(16):
                pltpu.async_copy(tbl.at[v[l]], rows_v.at[g * 16 + l], sem)
            return 0

        lax.fori_loop(0, _BPW // 16, step, 0)
        pltpu.make_async_copy(out.at[pl.ds(0, _BPW)], rows_v, sem).wait()
        pltpu.sync_copy(rows_v, out.at[pl.ds(base, _BPW)])


@functools.lru_cache(maxsize=1)
def _make_gather():
    return pl.kernel(
        _gather_body,
        mesh=plsc.VectorSubcoreMesh(core_axis_name="c", subcore_axis_name="s"),
        out_type=[
            jax.ShapeDtypeStruct((B, EMB), jnp.float32),
            jax.ShapeDtypeStruct((B, EMB), jnp.float32),
        ],
        scratch_types=[
            pltpu.VMEM((_BPW,), jnp.int32),
            pltpu.VMEM((_BPW, EMB), jnp.float32),
            pltpu.SemaphoreType.DMA,
        ],
    )


_BM = 512


def _unpack_half(words_f32, take_hi):
    w = jax.lax.bitcast_convert_type(words_f32, jnp.uint32)
    lo = w << 16
    hi = w & jnp.uint32(0xFFFF0000)
    return jax.lax.bitcast_convert_type(
        jnp.where(take_hi, hi, lo), jnp.float32)


def _mm_body(pp_ref, u_ref, i_ref, wu_ref, wi_ref, o_ref):
    pp = pp_ref[...]
    u_hi = (pp & 1) == 1
    i_hi = ((pp >> 1) & 1) == 1
    u_t = jnp.transpose(_unpack_half(u_ref[...], u_hi))
    i_t = jnp.transpose(_unpack_half(i_ref[...], i_hi))
    o_ref[...] = (
        jnp.dot(wu_ref[...], u_t, preferred_element_type=jnp.float32)
        + jnp.dot(wi_ref[...], i_t, preferred_element_type=jnp.float32)
    )


def _matmul(pp, u2, i2, wu, wi):
    return pl.pallas_call(
        _mm_body,
        grid=(B // _BM,),
        in_specs=[
            pl.BlockSpec((_BM, 1), lambda m: (m, 0)),
            pl.BlockSpec((_BM, EMB), lambda m: (m, 0)),
            pl.BlockSpec((_BM, EMB), lambda m: (m, 0)),
            pl.BlockSpec((VOCAB, EMB), lambda m: (0, 0)),
            pl.BlockSpec((VOCAB, EMB), lambda m: (0, 0)),
        ],
        out_specs=pl.BlockSpec((VOCAB, _BM), lambda m: (0, m)),
        out_shape=jax.ShapeDtypeStruct((VOCAB, B), jnp.float32),
    )(pp, u2, i2, wu, wi)


def kernel(attr_item, attr_tf_item, attr_lens_item, item_ids, attr_user,
           attr_tf_user, attr_lens_user, user_ids, pos_targets, pos_lens,
           neg_targets, neg_lens, user_table, item_table, W_user, W_item):
    uids = user_ids.astype(jnp.int32)
    iids = item_ids.astype(jnp.int32)
    u_hi = (uids >= H).astype(jnp.int32)
    i_hi = (iids >= H).astype(jnp.int32)
    uv = (uids - u_hi * H).reshape(_NW, _BPW)
    iv = (iids - i_hi * H).reshape(_NW, _BPW)
    u_tbl = _repack(user_table.T)
    i_tbl = _repack(item_table.T)
    u2, i2 = _make_gather()(u_tbl, uv, i_tbl, iv)
    pp = (u_hi + 2 * i_hi).reshape(B, 1)
    logits_t = _matmul(pp, u2, i2, W_user, W_item)
    return (jnp.transpose(logits_t), None, None)


# speedup vs baseline: 2.9824x; 1.0318x over previous
import functools

import jax
import jax.numpy as jnp
from jax import lax
from jax.experimental import pallas as pl
from jax.experimental.pallas import tpu as pltpu
from jax.experimental.pallas import tpu_sc as plsc

B = 16384
EMB = 64
VOCAB = 1000
NROWS = 1000000
VIEW_W = 2 * EMB

_BN = 8192
_NBLK = 62
H = _BN * _NBLK
_IN_BLOCKS = (NROWS + _BN - 1) // _BN - 1

_NC = 2
_NS = 16
_NW = _NC * _NS
_BPW = B // _NW


def _rp_body(a_ref, b_ref, o_ref):
    a16 = jax.lax.bitcast_convert_type(
        jnp.transpose(a_ref[...]).astype(jnp.bfloat16), jnp.uint16)
    b16 = jax.lax.bitcast_convert_type(
        jnp.transpose(b_ref[...]).astype(jnp.bfloat16), jnp.uint16)
    word = a16.astype(jnp.uint32) | (b16.astype(jnp.uint32) << 16)
    o_ref[...] = jax.lax.bitcast_convert_type(word, jnp.float32)


def _repack(tbl_t):
    return pl.pallas_call(
        _rp_body,
        grid=(_NBLK,),
        in_specs=[
            pl.BlockSpec((EMB, _BN), lambda n: (0, n)),
            pl.BlockSpec((EMB, _BN),
                         lambda n: (0, jnp.minimum(n + _NBLK, _IN_BLOCKS))),
        ],
        out_specs=pl.BlockSpec((_BN, EMB), lambda n: (n, 0)),
        out_shape=jax.ShapeDtypeStruct((H, EMB), jnp.float32),
    )(tbl_t, tbl_t)


def _gather_body(tbl, idx_hbm, out, idx_v, rows_v, sem):
    wid = lax.axis_index("s") * _NC + lax.axis_index("c")
    base = wid * _BPW
    pltpu.sync_copy(idx_hbm.at[wid], idx_v)

    def step(g, _):
        v = idx_v[pl.ds(g * 16, 16)]
        for l in range(16):
            pltpu.async_copy(tbl.at[v[l]], rows_v.at[g * 16 + l], sem)
        return 0

    lax.fori_loop(0, _BPW // 16, step, 0)
    pltpu.make_async_copy(out.at[pl.ds(0, _BPW)], rows_v, sem).wait()
    pltpu.sync_copy(rows_v, out.at[pl.ds(base, _BPW)])


@functools.lru_cache(maxsize=1)
def _make_gather():
    return pl.kernel(
        _gather_body,
        mesh=plsc.VectorSubcoreMesh(core_axis_name="c", subcore_axis_name="s"),
        out_type=jax.ShapeDtypeStruct((B, EMB), jnp.float32),
        scratch_types=[
            pltpu.VMEM((_BPW,), jnp.int32),
            pltpu.VMEM((_BPW, EMB), jnp.float32),
            pltpu.SemaphoreType.DMA,
        ],
    )


_BM = 1024


def _unpack_half(words_f32, take_hi):
    w = jax.lax.bitcast_convert_type(words_f32, jnp.uint32)
    lo = w << 16
    hi = w & jnp.uint32(0xFFFF0000)
    return jax.lax.bitcast_convert_type(
        jnp.where(take_hi, hi, lo), jnp.float32)


def _mm_body(pp_ref, u_ref, i_ref, wu_ref, wi_ref, o_ref):
    pp = pp_ref[...]
    u_hi = (pp & 1) == 1
    i_hi = ((pp >> 1) & 1) == 1
    u_t = jnp.transpose(_unpack_half(u_ref[...], u_hi))
    i_t = jnp.transpose(_unpack_half(i_ref[...], i_hi))
    o_ref[...] = (
        jnp.dot(wu_ref[...], u_t, preferred_element_type=jnp.float32)
        + jnp.dot(wi_ref[...], i_t, preferred_element_type=jnp.float32)
    )


def _matmul(pp, u2, i2, wu, wi):
    return pl.pallas_call(
        _mm_body,
        grid=(B // _BM,),
        in_specs=[
            pl.BlockSpec((_BM, 1), lambda m: (m, 0)),
            pl.BlockSpec((_BM, EMB), lambda m: (m, 0)),
            pl.BlockSpec((_BM, EMB), lambda m: (m, 0)),
            pl.BlockSpec((VOCAB, EMB), lambda m: (0, 0)),
            pl.BlockSpec((VOCAB, EMB), lambda m: (0, 0)),
        ],
        out_specs=pl.BlockSpec((VOCAB, _BM), lambda m: (0, m)),
        out_shape=jax.ShapeDtypeStruct((VOCAB, B), jnp.float32),
    )(pp, u2, i2, wu, wi)


def kernel(attr_item, attr_tf_item, attr_lens_item, item_ids, attr_user,
           attr_tf_user, attr_lens_user, user_ids, pos_targets, pos_lens,
           neg_targets, neg_lens, user_table, item_table, W_user, W_item):
    uids = user_ids.astype(jnp.int32)
    iids = item_ids.astype(jnp.int32)
    u_hi = (uids >= H).astype(jnp.int32)
    i_hi = (iids >= H).astype(jnp.int32)
    uv = (uids - u_hi * H).reshape(_NW, _BPW)
    iv = (iids - i_hi * H).reshape(_NW, _BPW)
    u_tbl = _repack(user_table.T)
    u2 = _make_gather()(u_tbl, uv)
    i_tbl = _repack(item_table.T)
    i2 = _make_gather()(i_tbl, iv)
    pp = (u_hi + 2 * i_hi).reshape(B, 1)
    logits_t = _matmul(pp, u2, i2, W_user, W_item)
    return (jnp.transpose(logits_t), None, None)


# speedup vs baseline: 3.0870x; 1.0351x over previous
import functools

import jax
import jax.numpy as jnp
from jax import lax
from jax.experimental import pallas as pl
from jax.experimental.pallas import tpu as pltpu
from jax.experimental.pallas import tpu_sc as plsc

B = 16384
EMB = 64
VOCAB = 1000
NROWS = 1000000
VIEW_W = 2 * EMB

_BN = 16384
_NBLK = 31
H = _BN * _NBLK
_IN_BLOCKS = (NROWS + _BN - 1) // _BN - 1

_NC = 2
_NS = 16
_NW = _NC * _NS
_BPW = B // _NW


def _rp_body(a_ref, b_ref, o_ref):
    a16 = jax.lax.bitcast_convert_type(
        jnp.transpose(a_ref[...]).astype(jnp.bfloat16), jnp.uint16)
    b16 = jax.lax.bitcast_convert_type(
        jnp.transpose(b_ref[...]).astype(jnp.bfloat16), jnp.uint16)
    word = a16.astype(jnp.uint32) | (b16.astype(jnp.uint32) << 16)
    o_ref[...] = jax.lax.bitcast_convert_type(word, jnp.float32)


def _repack(tbl_t):
    return pl.pallas_call(
        _rp_body,
        grid=(_NBLK,),
        in_specs=[
            pl.BlockSpec((EMB, _BN), lambda n: (0, n)),
            pl.BlockSpec((EMB, _BN),
                         lambda n: (0, jnp.minimum(n + _NBLK, _IN_BLOCKS))),
        ],
        out_specs=pl.BlockSpec((_BN, EMB), lambda n: (n, 0)),
        out_shape=jax.ShapeDtypeStruct((H, EMB), jnp.float32),
    )(tbl_t, tbl_t)


def _gather_body(tbl, idx_hbm, out, idx_v, rows_v, sem):
    wid = lax.axis_index("s") * _NC + lax.axis_index("c")
    base = wid * _BPW
    pltpu.sync_copy(idx_hbm.at[wid], idx_v)

    def step(g, _):
        v = idx_v[pl.ds(g * 16, 16)]
        for l in range(16):
            pltpu.async_copy(tbl.at[v[l]], rows_v.at[g * 16 + l], sem)
        return 0

    lax.fori_loop(0, _BPW // 16, step, 0)
    pltpu.make_async_copy(out.at[pl.ds(0, _BPW)], rows_v, sem).wait()
    pltpu.sync_copy(rows_v, out.at[pl.ds(base, _BPW)])


@functools.lru_cache(maxsize=1)
def _make_gather():
    return pl.kernel(
        _gather_body,
        mesh=plsc.VectorSubcoreMesh(core_axis_name="c", subcore_axis_name="s"),
        out_type=jax.ShapeDtypeStruct((B, EMB), jnp.float32),
        scratch_types=[
            pltpu.VMEM((_BPW,), jnp.int32),
            pltpu.VMEM((_BPW, EMB), jnp.float32),
            pltpu.SemaphoreType.DMA,
        ],
    )


_BM = 1024


def _unpack_half(words_f32, take_hi):
    w = jax.lax.bitcast_convert_type(words_f32, jnp.uint32)
    lo = w << 16
    hi = w & jnp.uint32(0xFFFF0000)
    return jax.lax.bitcast_convert_type(
        jnp.where(take_hi, hi, lo), jnp.float32)


def _mm_body(pp_ref, u_ref, i_ref, wu_ref, wi_ref, o_ref):
    pp = pp_ref[...]
    u_hi = (pp & 1) == 1
    i_hi = ((pp >> 1) & 1) == 1
    u_t = jnp.transpose(_unpack_half(u_ref[...], u_hi))
    i_t = jnp.transpose(_unpack_half(i_ref[...], i_hi))
    o_ref[...] = (
        jnp.dot(wu_ref[...], u_t, preferred_element_type=jnp.float32)
        + jnp.dot(wi_ref[...], i_t, preferred_element_type=jnp.float32)
    )


def _matmul(pp, u2, i2, wu, wi):
    return pl.pallas_call(
        _mm_body,
        grid=(B // _BM,),
        in_specs=[
            pl.BlockSpec((_BM, 1), lambda m: (m, 0)),
            pl.BlockSpec((_BM, EMB), lambda m: (m, 0)),
            pl.BlockSpec((_BM, EMB), lambda m: (m, 0)),
            pl.BlockSpec((VOCAB, EMB), lambda m: (0, 0)),
            pl.BlockSpec((VOCAB, EMB), lambda m: (0, 0)),
        ],
        out_specs=pl.BlockSpec((VOCAB, _BM), lambda m: (0, m)),
        out_shape=jax.ShapeDtypeStruct((VOCAB, B), jnp.float32),
    )(pp, u2, i2, wu, wi)


def kernel(attr_item, attr_tf_item, attr_lens_item, item_ids, attr_user,
           attr_tf_user, attr_lens_user, user_ids, pos_targets, pos_lens,
           neg_targets, neg_lens, user_table, item_table, W_user, W_item):
    uids = user_ids.astype(jnp.int32)
    iids = item_ids.astype(jnp.int32)
    u_hi = (uids >= H).astype(jnp.int32)
    i_hi = (iids >= H).astype(jnp.int32)
    uv = (uids - u_hi * H).reshape(_NW, _BPW)
    iv = (iids - i_hi * H).reshape(_NW, _BPW)
    u_tbl = _repack(user_table.T)
    u2 = _make_gather()(u_tbl, uv)
    i_tbl = _repack(item_table.T)
    i2 = _make_gather()(i_tbl, iv)
    pp = (u_hi + 2 * i_hi).reshape(B, 1)
    logits_t = _matmul(pp, u2, i2, W_user, W_item)
    return (jnp.transpose(logits_t), None, None)
